# Initial kernel scaffold; baseline (speedup 1.0000x reference)
import jax
import jax.numpy as jnp
from jax.experimental import pallas as pl


def kernel(x, edge_index, ctx, W1, b1, Wg1, bg1, Wb1, W2, b2, Wg2, bg2, Wb2):
    raise NotImplementedError("write your pallas kernel here")


# speedup vs baseline: 2.6142x; 2.6142x over previous
import jax
import jax.numpy as jnp
from jax.experimental import pallas as pl

N = 10000
E = 320000


def _gcn(x, edge_index, W, b, deginv):
    xl = (x * deginv[:, None]) @ W
    src = edge_index[0]
    dst = edge_index[1]
    agg = jax.ops.segment_sum(xl[src], dst, num_segments=N)
    return deginv[:, None] * (agg + xl) + b


def _add_kernel(a_ref, b_ref, o_ref):
    o_ref[...] = a_ref[...] + b_ref[...]


def kernel(x, edge_index, ctx, W1, b1, Wg1, bg1, Wb1, W2, b2, Wg2, bg2, Wb2):
    deg = jax.ops.segment_sum(jnp.ones((E,), jnp.float32), edge_index[1],
                              num_segments=N) + 1.0
    deginv = jax.lax.rsqrt(deg)
    g1 = jax.nn.sigmoid(ctx @ Wg1 + bg1)
    be1 = ctx @ Wb1
    g2 = jax.nn.sigmoid(ctx @ Wg2 + bg2)
    be2 = ctx @ Wb2
    h = _gcn(x, edge_index, W1, b1, deginv) * g1 + be1
    h = jax.nn.leaky_relu(h, negative_slope=0.01)
    out = _gcn(h, edge_index, W2, b2, deginv) * g2 + be2
    return pl.pallas_call(
        _add_kernel,
        out_shape=jax.ShapeDtypeStruct(x.shape, x.dtype),
    )(x, out)


# speedup vs baseline: 11.3976x; 4.3599x over previous
import functools

import jax
import jax.numpy as jnp
from jax import lax
from jax.experimental import pallas as pl
from jax.experimental.pallas import tpu as pltpu
from jax.experimental.pallas import tpu_sc as plsc

N = 10000
E = 320000
DIN = 128
DLAT = 256
NC = 2
NS = 16
NPT = 640
NPT_LAST = N - (NS - 1) * NPT
CH = 128
BN = 1000


def _per_tile_rows(s, do_copy):
    @pl.when(s < NS - 1)
    def _():
        do_copy(pl.multiple_of(s * NPT, 8), NPT)

    @pl.when(s == NS - 1)
    def _():
        do_copy((NS - 1) * NPT, NPT_LAST)


def _core_branch(c, fn):
    for cc in range(NC):
        @pl.when(c == cc)
        def _(cc=cc):
            fn(cc)

_MESH = plsc.VectorSubcoreMesh(
    core_axis_name="c", subcore_axis_name="s", num_cores=NC, num_subcores=NS)


def _deg_body(dst_hbm, zeros_hbm, ones_hbm, degp_hbm, dst_v, dst2_v, ones_v,
              acc_sh):
    c = lax.axis_index("c")
    s = lax.axis_index("s")
    w = c * NS + s
    _per_tile_rows(s, lambda off, nr: pltpu.sync_copy(
        zeros_hbm.at[pl.ds(off, nr)], acc_sh.at[pl.ds(off, nr)]))
    pltpu.sync_copy(ones_hbm, ones_v)
    plsc.subcore_barrier()
    epp = E // (NC * NS)
    ebase = w * epp
    nfull, rem = epp // CH, epp % CH

    def chunk(k, carry):
        pltpu.sync_copy(dst_hbm.at[pl.ds(ebase + k * CH, CH)], dst_v.at[0])
        pltpu.sync_copy(ones_v, acc_sh.at[dst_v.at[0]], add=True)
        return carry

    lax.fori_loop(0, nfull, chunk, 0)
    pltpu.sync_copy(dst_hbm.at[pl.ds(ebase + nfull * CH, rem)], dst2_v.at[0])
    pltpu.sync_copy(ones_v.at[pl.ds(0, rem)], acc_sh.at[dst2_v.at[0]],
                    add=True)
    plsc.subcore_barrier()
    _core_branch(c, lambda cc: _per_tile_rows(s, lambda off, nr: pltpu.sync_copy(
        acc_sh.at[pl.ds(off, nr)], degp_hbm.at[cc, pl.ds(off, nr)])))


_deg_call = functools.partial(
    pl.kernel,
    out_type=jax.ShapeDtypeStruct((NC, N, 128), jnp.float32),
    mesh=_MESH,
    scratch_types=[
        pltpu.VMEM((1, CH), jnp.int32),
        pltpu.VMEM((1, E // (NC * NS) % CH), jnp.int32),
        pltpu.VMEM((CH, 128), jnp.float32),
        pltpu.VMEM_SHARED((N, 128), jnp.float32),
    ],
)(_deg_body)


def _seg_chunks(ebase, nfull, rem, tab, src_hbm, dst_hbm,
                idx_v, dst_v, rows_v, idx2_v, dst2_v, rows2_v, acc_sh, sem):
    def chunk(k, carry):
        b = ebase + k * CH
        pltpu.sync_copy(src_hbm.at[pl.ds(b, CH)], idx_v.at[0])
        pltpu.sync_copy(dst_hbm.at[pl.ds(b, CH)], dst_v.at[0])
        pltpu.async_copy(tab.at[idx_v.at[0]], rows_v, sem).wait()
        pltpu.sync_copy(rows_v, acc_sh.at[dst_v.at[0]], add=True)
        return carry

    lax.fori_loop(0, nfull, chunk, 0)
    b = ebase + nfull * CH
    pltpu.sync_copy(src_hbm.at[pl.ds(b, rem)], idx2_v.at[0])
    pltpu.sync_copy(dst_hbm.at[pl.ds(b, rem)], dst2_v.at[0])
    pltpu.async_copy(tab.at[idx2_v.at[0]], rows2_v, sem).wait()
    pltpu.sync_copy(rows2_v, acc_sh.at[dst2_v.at[0]], add=True)


def _seg1_body(src_hbm, dst_hbm, y1b_hbm, zeros_hbm, raw1_hbm,
               idx_v, dst_v, rows_v, idx2_v, dst2_v, rows2_v, acc_sh, sem):
    c = lax.axis_index("c")
    s = lax.axis_index("s")
    epp = E // NS
    _per_tile_rows(s, lambda off, nr: pltpu.sync_copy(
        zeros_hbm.at[pl.ds(off, nr)], acc_sh.at[pl.ds(off, nr)]))
    plsc.subcore_barrier()
    _core_branch(c, lambda cc: _seg_chunks(
        s * epp, epp // CH, epp % CH, y1b_hbm.at[cc], src_hbm, dst_hbm,
        idx_v, dst_v, rows_v, idx2_v, dst2_v, rows2_v, acc_sh, sem))
    plsc.subcore_barrier()
    _core_branch(c, lambda cc: _per_tile_rows(s, lambda off, nr: pltpu.sync_copy(
        acc_sh.at[pl.ds(off, nr)], raw1_hbm.at[cc, pl.ds(off, nr)])))


def _seg2_body(src_hbm, dst_hbm, y2_hbm, zeros_hbm, raw2_hbm,
               idx_v, dst_v, rows_v, idx2_v, dst2_v, rows2_v, acc_sh, sem):
    c = lax.axis_index("c")
    s = lax.axis_index("s")
    epp = E // (NC * NS)
    _per_tile_rows(s, lambda off, nr: pltpu.sync_copy(
        zeros_hbm.at[pl.ds(off, nr)], acc_sh.at[pl.ds(off, nr)]))
    plsc.subcore_barrier()
    _core_branch(c, lambda cc: _seg_chunks(
        (cc * NS + s) * epp, epp // CH, epp % CH, y2_hbm, src_hbm, dst_hbm,
        idx_v, dst_v, rows_v, idx2_v, dst2_v, rows2_v, acc_sh, sem))
    plsc.subcore_barrier()
    _core_branch(c, lambda cc: _per_tile_rows(s, lambda off, nr: pltpu.sync_copy(
        acc_sh.at[pl.ds(off, nr)], raw2_hbm.at[cc, pl.ds(off, nr)])))


def _seg_scratch(rem):
    return [
        pltpu.VMEM((1, CH), jnp.int32),
        pltpu.VMEM((1, CH), jnp.int32),
        pltpu.VMEM((CH, 128), jnp.float32),
        pltpu.VMEM((1, rem), jnp.int32),
        pltpu.VMEM((1, rem), jnp.int32),
        pltpu.VMEM((rem, 128), jnp.float32),
        pltpu.VMEM_SHARED((N, 128), jnp.float32),
        pltpu.SemaphoreType.DMA,
    ]


_seg1_call = functools.partial(
    pl.kernel,
    out_type=jax.ShapeDtypeStruct((NC, N, 128), jnp.float32),
    mesh=_MESH,
    scratch_types=_seg_scratch((E // NS) % CH),
)(_seg1_body)

_seg2_call = functools.partial(
    pl.kernel,
    out_type=jax.ShapeDtypeStruct((NC, N, 128), jnp.float32),
    mesh=_MESH,
    scratch_types=_seg_scratch((E // (NC * NS)) % CH),
)(_seg2_body)


def _dinv_of(degp_ref):
    d = degp_ref[0][:, :1] + degp_ref[1][:, :1] + 1.0
    return jax.lax.rsqrt(d)


def _gates_body(ctx_ref, wg1_ref, bg1_ref, wb1_ref, wg2_ref, bg2_ref, wb2_ref,
                g1_ref, be1_ref, g2_ref, be2_ref):
    ctx = ctx_ref[...]
    g1_ref[...] = jax.nn.sigmoid(
        jnp.dot(ctx, wg1_ref[...], preferred_element_type=jnp.float32)
        + bg1_ref[...])
    be1_ref[...] = jnp.dot(ctx, wb1_ref[...], preferred_element_type=jnp.float32)
    g2_ref[...] = jax.nn.sigmoid(
        jnp.dot(ctx, wg2_ref[...], preferred_element_type=jnp.float32)
        + bg2_ref[...])
    be2_ref[...] = jnp.dot(ctx, wb2_ref[...], preferred_element_type=jnp.float32)


def _gates_call(ctx, Wg1, bg1, Wb1, Wg2, bg2, Wb2):
    return pl.pallas_call(
        _gates_body,
        out_shape=(
            jax.ShapeDtypeStruct((1, DLAT), jnp.float32),
            jax.ShapeDtypeStruct((1, DLAT), jnp.float32),
            jax.ShapeDtypeStruct((1, DIN), jnp.float32),
            jax.ShapeDtypeStruct((1, DIN), jnp.float32),
        ),
    )(ctx, Wg1, bg1.reshape(1, DLAT), Wb1, Wg2, bg2.reshape(1, DIN), Wb2)


def _mm1_body(x_ref, degp_ref, w1_ref, o_ref):
    dinv = _dinv_of(degp_ref)
    t = x_ref[...] * dinv
    o_ref[0] = jnp.dot(t, w1_ref[...], preferred_element_type=jnp.float32)


def _mm1_call(x, degp, W1):
    nb = N // BN
    return pl.pallas_call(
        _mm1_body,
        grid=(nb, 2),
        in_specs=[
            pl.BlockSpec((BN, DIN), lambda i, c: (i, 0)),
            pl.BlockSpec((NC, BN, 128), lambda i, c: (0, i, 0)),
            pl.BlockSpec((DIN, 128), lambda i, c: (0, c)),
        ],
        out_specs=pl.BlockSpec((1, BN, 128), lambda i, c: (c, i, 0)),
        out_shape=jax.ShapeDtypeStruct((2, N, 128), jnp.float32),
    )(x, degp, W1)


def _mm2_body(raw1_ref, y1b_ref, degp_ref, b1_ref, g1_ref, be1_ref, w2_ref,
              o_ref):
    c = pl.program_id(1)
    dinv = _dinv_of(degp_ref)
    t = dinv * (raw1_ref[0] + y1b_ref[0]) + b1_ref[0]
    h = t * g1_ref[0] + be1_ref[0]
    h = jnp.where(h > 0, h, 0.01 * h)
    contrib = jnp.dot(h * dinv, w2_ref[...], preferred_element_type=jnp.float32)

    @pl.when(c == 0)
    def _():
        o_ref[...] = contrib

    @pl.when(c == 1)
    def _():
        o_ref[...] += contrib


def _mm2_call(raw1, y1b, degp, b1r, g1r, be1r, W2):
    nb = N // BN
    return pl.pallas_call(
        _mm2_body,
        grid=(nb, 2),
        in_specs=[
            pl.BlockSpec((1, BN, 128), lambda i, c: (c, i, 0)),
            pl.BlockSpec((1, BN, 128), lambda i, c: (c, i, 0)),
            pl.BlockSpec((NC, BN, 128), lambda i, c: (0, i, 0)),
            pl.BlockSpec((1, 1, 128), lambda i, c: (c, 0, 0)),
            pl.BlockSpec((1, 1, 128), lambda i, c: (c, 0, 0)),
            pl.BlockSpec((1, 1, 128), lambda i, c: (c, 0, 0)),
            pl.BlockSpec((128, DIN), lambda i, c: (c, 0)),
        ],
        out_specs=pl.BlockSpec((BN, DIN), lambda i, c: (i, 0)),
        out_shape=jax.ShapeDtypeStruct((N, DIN), jnp.float32),
    )(raw1, y1b, degp, b1r, g1r, be1r, W2)


def _final_body(raw2_ref, y2_ref, x_ref, degp_ref, b2_ref, g2_ref, be2_ref,
                o_ref):
    dinv = _dinv_of(degp_ref)
    t = dinv * (raw2_ref[0] + raw2_ref[1] + y2_ref[...]) + b2_ref[...]
    o_ref[...] = x_ref[...] + t * g2_ref[...] + be2_ref[...]


def _final_call(raw2, y2, x, degp, b2r, g2r, be2r):
    nb = N // BN
    return pl.pallas_call(
        _final_body,
        grid=(nb,),
        in_specs=[
            pl.BlockSpec((NC, BN, 128), lambda i: (0, i, 0)),
            pl.BlockSpec((BN, DIN), lambda i: (i, 0)),
            pl.BlockSpec((BN, DIN), lambda i: (i, 0)),
            pl.BlockSpec((NC, BN, 128), lambda i: (0, i, 0)),
            pl.BlockSpec((1, DIN), lambda i: (0, 0)),
            pl.BlockSpec((1, DIN), lambda i: (0, 0)),
            pl.BlockSpec((1, DIN), lambda i: (0, 0)),
        ],
        out_specs=pl.BlockSpec((BN, DIN), lambda i: (i, 0)),
        out_shape=jax.ShapeDtypeStruct((N, DIN), jnp.float32),
    )(raw2, y2, x, degp, b2r, g2r, be2r)


def kernel(x, edge_index, ctx, W1, b1, Wg1, bg1, Wb1, W2, b2, Wg2, bg2, Wb2):
    src = edge_index[0]
    dst = edge_index[1]
    zerosN = jnp.zeros((N, 128), jnp.float32)
    onesC = jnp.ones((CH, 128), jnp.float32)

    degp = _deg_call(dst, zerosN, onesC)
    g1, be1, g2, be2 = _gates_call(ctx, Wg1, bg1, Wb1, Wg2, bg2, Wb2)
    y1b = _mm1_call(x, degp, W1)
    raw1 = _seg1_call(src, dst, y1b, zerosN)
    y2 = _mm2_call(raw1, y1b, degp,
                   b1.reshape(2, 1, 128), g1.reshape(2, 1, 128),
                   be1.reshape(2, 1, 128), W2)
    raw2 = _seg2_call(src, dst, y2, zerosN)
    return _final_call(raw2, y2, x, degp,
                       b2.reshape(1, 128), g2.reshape(1, 128),
                       be2.reshape(1, 128))


# speedup vs baseline: 17.4704x; 1.5328x over previous
import functools

import jax
import jax.numpy as jnp
from jax import lax
from jax.experimental import pallas as pl
from jax.experimental.pallas import tpu as pltpu
from jax.experimental.pallas import tpu_sc as plsc

N = 10000
E = 320000
DIN = 128
DLAT = 256
NC = 2
NS = 16
NPT = 640
NPT_LAST = N - (NS - 1) * NPT
CH = 128
BN = 1000


def _per_tile_rows(s, do_copy):
    @pl.when(s < NS - 1)
    def _():
        do_copy(pl.multiple_of(s * NPT, 8), NPT)

    @pl.when(s == NS - 1)
    def _():
        do_copy((NS - 1) * NPT, NPT_LAST)


def _core_branch(c, fn):
    for cc in range(NC):
        @pl.when(c == cc)
        def _(cc=cc):
            fn(cc)

_MESH = plsc.VectorSubcoreMesh(
    core_axis_name="c", subcore_axis_name="s", num_cores=NC, num_subcores=NS)


def _deg_body(dst_hbm, zeros_hbm, ones_hbm, degp_hbm, dst_v, dstb_v, dst2_v,
              ones_v, acc_sh, ss0, ss1):
    c = lax.axis_index("c")
    s = lax.axis_index("s")
    w = c * NS + s
    _per_tile_rows(s, lambda off, nr: pltpu.sync_copy(
        zeros_hbm.at[pl.ds(off, nr)], acc_sh.at[pl.ds(off, nr)]))
    pltpu.sync_copy(ones_hbm, ones_v)
    plsc.subcore_barrier()
    epp = E // (NC * NS)
    ebase = w * epp
    nfull, rem = epp // CH, epp % CH

    def load_dst(k, db):
        pltpu.sync_copy(dst_hbm.at[pl.ds(ebase + k * CH, CH)], db.at[0])

    def scatter(db, sem):
        pltpu.async_copy(ones_v, acc_sh.at[db.at[0]], sem, add=True)

    def wait_scatter(db, sem):
        pltpu.make_async_copy(ones_v, acc_sh.at[db.at[0]], sem).wait()

    load_dst(0, dst_v)

    def pair(k2, carry):
        ka = 2 * k2
        scatter(dst_v, ss0)

        @pl.when(k2 > 0)
        def _():
            wait_scatter(dstb_v, ss1)
        load_dst(ka + 1, dstb_v)
        scatter(dstb_v, ss1)
        wait_scatter(dst_v, ss0)

        @pl.when(ka + 2 < nfull)
        def _():
            load_dst(ka + 2, dst_v)
        return carry

    lax.fori_loop(0, nfull // 2, pair, 0)
    wait_scatter(dstb_v, ss1)
    pltpu.sync_copy(dst_hbm.at[pl.ds(ebase + nfull * CH, rem)], dst2_v.at[0])
    pltpu.sync_copy(ones_v.at[pl.ds(0, rem)], acc_sh.at[dst2_v.at[0]],
                    add=True)
    plsc.subcore_barrier()
    _core_branch(c, lambda cc: _per_tile_rows(s, lambda off, nr: pltpu.sync_copy(
        acc_sh.at[pl.ds(off, nr)], degp_hbm.at[cc, pl.ds(off, nr)])))


_deg_call = functools.partial(
    pl.kernel,
    out_type=jax.ShapeDtypeStruct((NC, N, 128), jnp.float32),
    mesh=_MESH,
    scratch_types=[
        pltpu.VMEM((1, CH), jnp.int32),
        pltpu.VMEM((1, CH), jnp.int32),
        pltpu.VMEM((1, E // (NC * NS) % CH), jnp.int32),
        pltpu.VMEM((CH, 128), jnp.float32),
        pltpu.VMEM_SHARED((N, 128), jnp.float32),
        pltpu.SemaphoreType.DMA,
        pltpu.SemaphoreType.DMA,
    ],
)(_deg_body)


def _seg_chunks(ebase, nfull, rem, tab, src_hbm, dst_hbm,
                i0, d0, r0, i1, d1, r1, idx2_v, dst2_v, rows2_v, acc_sh,
                sg0, sg1, ss0, ss1, s2):
    def load_idx(k, ib, db):
        pltpu.sync_copy(src_hbm.at[pl.ds(ebase + k * CH, CH)], ib.at[0])
        pltpu.sync_copy(dst_hbm.at[pl.ds(ebase + k * CH, CH)], db.at[0])

    def gather(ib, rb, sem):
        pltpu.async_copy(tab.at[ib.at[0]], rb, sem)

    def wait_gather(ib, rb, sem):
        pltpu.make_async_copy(tab.at[ib.at[0]], rb, sem).wait()

    def scatter(rb, db, sem):
        pltpu.async_copy(rb, acc_sh.at[db.at[0]], sem, add=True)

    def wait_scatter(rb, db, sem):
        pltpu.make_async_copy(rb, acc_sh.at[db.at[0]], sem).wait()

    load_idx(0, i0, d0)
    gather(i0, r0, sg0)

    def pair(k2, carry):
        ka = 2 * k2

        @pl.when(k2 > 0)
        def _():
            wait_scatter(r1, d1, ss1)
        load_idx(ka + 1, i1, d1)
        wait_gather(i0, r0, sg0)
        scatter(r0, d0, ss0)
        gather(i1, r1, sg1)
        wait_scatter(r0, d0, ss0)

        @pl.when(ka + 2 < nfull)
        def _():
            load_idx(ka + 2, i0, d0)
        wait_gather(i1, r1, sg1)
        scatter(r1, d1, ss1)

        @pl.when(ka + 2 < nfull)
        def _():
            gather(i0, r0, sg0)
        return carry

    lax.fori_loop(0, nfull // 2, pair, 0)
    wait_scatter(r1, d1, ss1)
    b = ebase + nfull * CH
    pltpu.sync_copy(src_hbm.at[pl.ds(b, rem)], idx2_v.at[0])
    pltpu.sync_copy(dst_hbm.at[pl.ds(b, rem)], dst2_v.at[0])
    pltpu.async_copy(tab.at[idx2_v.at[0]], rows2_v, s2).wait()
    pltpu.sync_copy(rows2_v, acc_sh.at[dst2_v.at[0]], add=True)


def _seg1_body(src_hbm, dst_hbm, y1b_hbm, zeros_hbm, raw1_hbm,
               i0, d0, r0, i1, d1, r1, idx2_v, dst2_v, rows2_v, acc_sh,
               sg0, sg1, ss0, ss1, s2):
    c = lax.axis_index("c")
    s = lax.axis_index("s")
    epp = E // NS
    _per_tile_rows(s, lambda off, nr: pltpu.sync_copy(
        zeros_hbm.at[pl.ds(off, nr)], acc_sh.at[pl.ds(off, nr)]))
    plsc.subcore_barrier()
    _core_branch(c, lambda cc: _seg_chunks(
        s * epp, epp // CH, epp % CH, y1b_hbm.at[cc], src_hbm, dst_hbm,
        i0, d0, r0, i1, d1, r1, idx2_v, dst2_v, rows2_v, acc_sh,
        sg0, sg1, ss0, ss1, s2))
    plsc.subcore_barrier()
    _core_branch(c, lambda cc: _per_tile_rows(s, lambda off, nr: pltpu.sync_copy(
        acc_sh.at[pl.ds(off, nr)], raw1_hbm.at[cc, pl.ds(off, nr)])))


def _seg2_body(src_hbm, dst_hbm, y2_hbm, zeros_hbm, raw2_hbm,
               i0, d0, r0, i1, d1, r1, idx2_v, dst2_v, rows2_v, acc_sh,
               sg0, sg1, ss0, ss1, s2):
    c = lax.axis_index("c")
    s = lax.axis_index("s")
    epp = E // (NC * NS)
    _per_tile_rows(s, lambda off, nr: pltpu.sync_copy(
        zeros_hbm.at[pl.ds(off, nr)], acc_sh.at[pl.ds(off, nr)]))
    plsc.subcore_barrier()
    _core_branch(c, lambda cc: _seg_chunks(
        (cc * NS + s) * epp, epp // CH, epp % CH, y2_hbm, src_hbm, dst_hbm,
        i0, d0, r0, i1, d1, r1, idx2_v, dst2_v, rows2_v, acc_sh,
        sg0, sg1, ss0, ss1, s2))
    plsc.subcore_barrier()
    _core_branch(c, lambda cc: _per_tile_rows(s, lambda off, nr: pltpu.sync_copy(
        acc_sh.at[pl.ds(off, nr)], raw2_hbm.at[cc, pl.ds(off, nr)])))


def _seg_scratch(rem):
    return [
        pltpu.VMEM((1, CH), jnp.int32),
        pltpu.VMEM((1, CH), jnp.int32),
        pltpu.VMEM((CH, 128), jnp.float32),
        pltpu.VMEM((1, CH), jnp.int32),
        pltpu.VMEM((1, CH), jnp.int32),
        pltpu.VMEM((CH, 128), jnp.float32),
        pltpu.VMEM((1, rem), jnp.int32),
        pltpu.VMEM((1, rem), jnp.int32),
        pltpu.VMEM((rem, 128), jnp.float32),
        pltpu.VMEM_SHARED((N, 128), jnp.float32),
        pltpu.SemaphoreType.DMA,
        pltpu.SemaphoreType.DMA,
        pltpu.SemaphoreType.DMA,
        pltpu.SemaphoreType.DMA,
        pltpu.SemaphoreType.DMA,
    ]


_seg1_call = functools.partial(
    pl.kernel,
    out_type=jax.ShapeDtypeStruct((NC, N, 128), jnp.float32),
    mesh=_MESH,
    scratch_types=_seg_scratch((E // NS) % CH),
)(_seg1_body)

_seg2_call = functools.partial(
    pl.kernel,
    out_type=jax.ShapeDtypeStruct((NC, N, 128), jnp.float32),
    mesh=_MESH,
    scratch_types=_seg_scratch((E // (NC * NS)) % CH),
)(_seg2_body)


def _dinv_of(degp_ref):
    d = degp_ref[0][:, :1] + degp_ref[1][:, :1] + 1.0
    return jax.lax.rsqrt(d)


def _gates_body(ctx_ref, wg1_ref, bg1_ref, wb1_ref, wg2_ref, bg2_ref, wb2_ref,
                g1_ref, be1_ref, g2_ref, be2_ref):
    ctx = ctx_ref[...]
    g1_ref[...] = jax.nn.sigmoid(
        jnp.dot(ctx, wg1_ref[...], preferred_element_type=jnp.float32)
        + bg1_ref[...])
    be1_ref[...] = jnp.dot(ctx, wb1_ref[...], preferred_element_type=jnp.float32)
    g2_ref[...] = jax.nn.sigmoid(
        jnp.dot(ctx, wg2_ref[...], preferred_element_type=jnp.float32)
        + bg2_ref[...])
    be2_ref[...] = jnp.dot(ctx, wb2_ref[...], preferred_element_type=jnp.float32)


def _gates_call(ctx, Wg1, bg1, Wb1, Wg2, bg2, Wb2):
    return pl.pallas_call(
        _gates_body,
        out_shape=(
            jax.ShapeDtypeStruct((1, DLAT), jnp.float32),
            jax.ShapeDtypeStruct((1, DLAT), jnp.float32),
            jax.ShapeDtypeStruct((1, DIN), jnp.float32),
            jax.ShapeDtypeStruct((1, DIN), jnp.float32),
        ),
    )(ctx, Wg1, bg1.reshape(1, DLAT), Wb1, Wg2, bg2.reshape(1, DIN), Wb2)


def _mm1_body(x_ref, degp_ref, w1_ref, o_ref):
    dinv = _dinv_of(degp_ref)
    t = x_ref[...] * dinv
    o_ref[0] = jnp.dot(t, w1_ref[...], preferred_element_type=jnp.float32)


def _mm1_call(x, degp, W1):
    nb = N // BN
    return pl.pallas_call(
        _mm1_body,
        grid=(nb, 2),
        in_specs=[
            pl.BlockSpec((BN, DIN), lambda i, c: (i, 0)),
            pl.BlockSpec((NC, BN, 128), lambda i, c: (0, i, 0)),
            pl.BlockSpec((DIN, 128), lambda i, c: (0, c)),
        ],
        out_specs=pl.BlockSpec((1, BN, 128), lambda i, c: (c, i, 0)),
        out_shape=jax.ShapeDtypeStruct((2, N, 128), jnp.float32),
    )(x, degp, W1)


def _mm2_body(raw1_ref, y1b_ref, degp_ref, b1_ref, g1_ref, be1_ref, w2_ref,
              o_ref):
    c = pl.program_id(1)
    dinv = _dinv_of(degp_ref)
    t = dinv * (raw1_ref[0] + y1b_ref[0]) + b1_ref[0]
    h = t * g1_ref[0] + be1_ref[0]
    h = jnp.where(h > 0, h, 0.01 * h)
    contrib = jnp.dot(h * dinv, w2_ref[...], preferred_element_type=jnp.float32)

    @pl.when(c == 0)
    def _():
        o_ref[...] = contrib

    @pl.when(c == 1)
    def _():
        o_ref[...] += contrib


def _mm2_call(raw1, y1b, degp, b1r, g1r, be1r, W2):
    nb = N // BN
    return pl.pallas_call(
        _mm2_body,
        grid=(nb, 2),
        in_specs=[
            pl.BlockSpec((1, BN, 128), lambda i, c: (c, i, 0)),
            pl.BlockSpec((1, BN, 128), lambda i, c: (c, i, 0)),
            pl.BlockSpec((NC, BN, 128), lambda i, c: (0, i, 0)),
            pl.BlockSpec((1, 1, 128), lambda i, c: (c, 0, 0)),
            pl.BlockSpec((1, 1, 128), lambda i, c: (c, 0, 0)),
            pl.BlockSpec((1, 1, 128), lambda i, c: (c, 0, 0)),
            pl.BlockSpec((128, DIN), lambda i, c: (c, 0)),
        ],
        out_specs=pl.BlockSpec((BN, DIN), lambda i, c: (i, 0)),
        out_shape=jax.ShapeDtypeStruct((N, DIN), jnp.float32),
    )(raw1, y1b, degp, b1r, g1r, be1r, W2)


def _final_body(raw2_ref, y2_ref, x_ref, degp_ref, b2_ref, g2_ref, be2_ref,
                o_ref):
    dinv = _dinv_of(degp_ref)
    t = dinv * (raw2_ref[0] + raw2_ref[1] + y2_ref[...]) + b2_ref[...]
    o_ref[...] = x_ref[...] + t * g2_ref[...] + be2_ref[...]


def _final_call(raw2, y2, x, degp, b2r, g2r, be2r):
    nb = N // BN
    return pl.pallas_call(
        _final_body,
        grid=(nb,),
        in_specs=[
            pl.BlockSpec((NC, BN, 128), lambda i: (0, i, 0)),
            pl.BlockSpec((BN, DIN), lambda i: (i, 0)),
            pl.BlockSpec((BN, DIN), lambda i: (i, 0)),
            pl.BlockSpec((NC, BN, 128), lambda i: (0, i, 0)),
            pl.BlockSpec((1, DIN), lambda i: (0, 0)),
            pl.BlockSpec((1, DIN), lambda i: (0, 0)),
            pl.BlockSpec((1, DIN), lambda i: (0, 0)),
        ],
        out_specs=pl.BlockSpec((BN, DIN), lambda i: (i, 0)),
        out_shape=jax.ShapeDtypeStruct((N, DIN), jnp.float32),
    )(raw2, y2, x, degp, b2r, g2r, be2r)


def kernel(x, edge_index, ctx, W1, b1, Wg1, bg1, Wb1, W2, b2, Wg2, bg2, Wb2):
    src = edge_index[0]
    dst = edge_index[1]
    zerosN = jnp.zeros((N, 128), jnp.float32)
    onesC = jnp.ones((CH, 128), jnp.float32)

    degp = _deg_call(dst, zerosN, onesC)
    g1, be1, g2, be2 = _gates_call(ctx, Wg1, bg1, Wb1, Wg2, bg2, Wb2)
    y1b = _mm1_call(x, degp, W1)
    raw1 = _seg1_call(src, dst, y1b, zerosN)
    y2 = _mm2_call(raw1, y1b, degp,
                   b1.reshape(2, 1, 128), g1.reshape(2, 1, 128),
                   be1.reshape(2, 1, 128), W2)
    raw2 = _seg2_call(src, dst, y2, zerosN)
    return _final_call(raw2, y2, x, degp,
                       b2.reshape(1, 128), g2.reshape(1, 128),
                       be2.reshape(1, 128))
